# baseline (speedup 1.0000x reference)
import functools
import math

import jax
import jax.numpy as jnp
from jax import lax
from jax.experimental import pallas as pl
from jax.experimental.pallas import tpu as pltpu
from jax.experimental.pallas import tpu_sc as plsc

_N = 10000
_NPAD = 10240
_D = 128
_K = 16
_RB = 256
_AB = 256


def _knn_body(n_valid, k, xr_ref, ct_ref, out_ref):
    rb = xr_ref.shape[0]
    npad = ct_ref.shape[1]
    b = pl.program_id(0)
    xr = xr_ref[...]
    lane8 = lax.broadcasted_iota(jnp.int32, (rb, 8), 1)
    rc = jnp.where(lane8 < 3, xr[:, :8], 0.0)
    ct = ct_ref[...]
    inf = jnp.float32(jnp.inf)

    ab = jnp.dot(rc, ct, preferred_element_type=jnp.float32)
    sqi = jnp.sum(rc * rc, axis=1, keepdims=True)
    sqj = jnp.sum(ct * ct, axis=0, keepdims=True)
    col = lax.broadcasted_iota(jnp.int32, (rb, npad), 1)
    row = b * rb + lax.broadcasted_iota(jnp.int32, (rb, npad), 0)
    d2 = sqi - 2.0 * ab + sqj
    d2 = jnp.where((col == row) | (col >= n_valid), inf, d2)

    outs = []
    for _ in range(k):
        idx = jnp.argmin(d2, axis=1).astype(jnp.int32)[:, None]
        outs.append(idx)
        d2 = jnp.where(col == idx, inf, d2)
    out_ref[...] = jnp.concatenate(outs, axis=1)


def _knn(xp, ct, n_valid, rb, k):
    npad, d = xp.shape
    return pl.pallas_call(
        functools.partial(_knn_body, n_valid, k),
        grid=(npad // rb,),
        in_specs=[
            pl.BlockSpec((rb, d), lambda i: (i, 0)),
            pl.BlockSpec((8, npad), lambda i: (0, 0)),
        ],
        out_specs=pl.BlockSpec((rb, k), lambda i: (i, 0)),
        out_shape=jax.ShapeDtypeStruct((npad, k), jnp.int32),
    )(xp, ct)


def _attn_body(scale_den, act, res, k,
               x_ref, xs_ref, wq_ref, bq_ref, wk_ref, bk_ref,
               wv_ref, bv_ref, ws_ref, bs_ref, o_ref):
    nb, d = x_ref.shape
    xb = x_ref[...]
    xs = xs_ref[...]
    q = jnp.dot(xb, wq_ref[...], preferred_element_type=jnp.float32) + bq_ref[...]
    ks = jnp.dot(xs, wk_ref[...], preferred_element_type=jnp.float32) + bk_ref[...]
    vs = jnp.dot(xs, wv_ref[...], preferred_element_type=jnp.float32) + bv_ref[...]
    k3 = ks.reshape(nb, k, d)
    v3 = vs.reshape(nb, k, d)
    logits = jnp.sum(k3 * q[:, None, :], axis=2) / scale_den
    m = jnp.max(logits, axis=1, keepdims=True)
    ex = jnp.exp(logits - m)
    den = jnp.sum(ex, axis=1, keepdims=True)
    alpha = ex / (den + 1e-16)
    agg = jnp.sum(alpha[:, :, None] * v3, axis=1)
    out = agg + jnp.dot(xb, ws_ref[...], preferred_element_type=jnp.float32) + bs_ref[...]
    if res:
        out = out + xb
    if act:
        out = jnp.tanh(out)
    o_ref[...] = out


def _attn(x, xs, wq, bq, wk, bk, wv, bv, ws, bs, ch, act, res, nb, k):
    npad, d = x.shape
    scale_den = float(math.sqrt(float(ch)))
    body = functools.partial(_attn_body, scale_den, act, res, k)
    w_spec = pl.BlockSpec((d, d), lambda i: (0, 0))
    b_spec = pl.BlockSpec((1, d), lambda i: (0, 0))
    return pl.pallas_call(
        body,
        grid=(npad // nb,),
        in_specs=[
            pl.BlockSpec((nb, d), lambda i: (i, 0)),
            pl.BlockSpec((nb * k, d), lambda i: (i, 0)),
            w_spec, b_spec, w_spec, b_spec, w_spec, b_spec, w_spec, b_spec,
        ],
        out_specs=pl.BlockSpec((nb, d), lambda i: (i, 0)),
        out_shape=jax.ShapeDtypeStruct((npad, d), jnp.float32),
    )(x, xs, wq, bq.reshape(1, d), wk, bk.reshape(1, d),
      wv, bv.reshape(1, d), ws, bs.reshape(1, d))


def _sc_gather(table, idx2d):
    npad, d = table.shape
    nrow = idx2d.shape[0]
    ne = nrow * 128
    info = plsc.get_sparse_core_info()
    nw = info.num_cores * info.num_subcores
    rpw = nrow // nw
    mesh = plsc.VectorSubcoreMesh(core_axis_name="c", subcore_axis_name="s")

    @functools.partial(
        pl.kernel,
        mesh=mesh,
        out_type=jax.ShapeDtypeStruct((ne, d), jnp.float32),
        scratch_types=[
            pltpu.VMEM((rpw, 128), jnp.int32),
            pltpu.VMEM((128, d), jnp.float32),
            pltpu.SemaphoreType.DMA,
        ],
    )
    def gk(table_hbm, idx_hbm, out_hbm, idx_v, rows_v, sem):
        widx = lax.axis_index("s") * info.num_cores + lax.axis_index("c")
        rbase = widx * rpw
        pltpu.sync_copy(idx_hbm.at[pl.ds(rbase, rpw)], idx_v)

        def body(j, carry):
            pltpu.async_copy(table_hbm.at[idx_v.at[j]], rows_v, sem).wait()
            pltpu.sync_copy(rows_v, out_hbm.at[pl.ds((rbase + j) * 128, 128)])
            return carry

        lax.fori_loop(0, rpw, body, 0)

    return gk(table, idx2d)


def _pad_w(w, d):
    wp = jnp.zeros((d, d), jnp.float32)
    return wp.at[:, : w.shape[1]].set(w)


def _pad_b(b, d):
    return jnp.zeros((d,), jnp.float32).at[: b.shape[0]].set(b)


def kernel(t, x,
           W1q, b1q, W1k, b1k, W1v, b1v, W1s, b1s,
           W2q, b2q, W2k, b2k, W2v, b2v, W2s, b2s,
           W3q, b3q, W3k, b3k, W3v, b3v, W3s, b3s):
    del t
    d = _D
    xp = jnp.pad(x, ((0, _NPAD - _N), (0, 0)))
    ct = jnp.zeros((8, _NPAD), jnp.float32).at[:3, :_N].set(x[:, :3].T)

    src = _knn(xp, ct, _N, _RB, _K)
    idx2d = src.reshape(_NPAD * _K // 128, 128)

    xs1 = _sc_gather(xp, idx2d)
    h1 = _attn(xp, xs1, W1q, b1q, W1k, b1k, W1v, b1v, W1s, b1s,
               ch=128, act=True, res=False, nb=_AB, k=_K)
    xs2 = _sc_gather(h1, idx2d)
    h2 = _attn(h1, xs2, W2q, b2q, W2k, b2k, W2v, b2v, W2s, b2s,
               ch=128, act=True, res=True, nb=_AB, k=_K)
    xs3 = _sc_gather(h2, idx2d)
    out = _attn(h2, xs3,
                _pad_w(W3q, d), _pad_b(b3q, d), _pad_w(W3k, d), _pad_b(b3k, d),
                _pad_w(W3v, d), _pad_b(b3v, d), _pad_w(W3s, d), _pad_b(b3s, d),
                ch=3, act=False, res=False, nb=_AB, k=_K)
    return out[:_N, :3]


# speedup vs baseline: 1.0303x; 1.0303x over previous
import functools
import math

import jax
import jax.numpy as jnp
from jax import lax
from jax.experimental import pallas as pl
from jax.experimental.pallas import tpu as pltpu
from jax.experimental.pallas import tpu_sc as plsc

_N = 10000
_NPAD = 10240
_D = 128
_K = 16
_RB = 256
_AB = 256


def _knn_body(n_valid, k, xr_ref, ct_ref, out_ref):
    rb = xr_ref.shape[0]
    npad = ct_ref.shape[1]
    b = pl.program_id(0)
    xr = xr_ref[...]
    lane8 = lax.broadcasted_iota(jnp.int32, (rb, 8), 1)
    rc = jnp.where(lane8 < 3, xr[:, :8], 0.0)
    ct = ct_ref[...]
    inf = jnp.float32(jnp.inf)

    ab = jnp.dot(rc, ct, preferred_element_type=jnp.float32)
    sqi = jnp.sum(rc * rc, axis=1, keepdims=True)
    sqj = jnp.sum(ct * ct, axis=0, keepdims=True)
    col = lax.broadcasted_iota(jnp.int32, (rb, npad), 1)
    row = b * rb + lax.broadcasted_iota(jnp.int32, (rb, npad), 0)
    d2 = sqi - 2.0 * ab + sqj
    d2 = jnp.where((col == row) | (col >= n_valid), inf, d2)

    outs = []
    for _ in range(k):
        idx = jnp.argmin(d2, axis=1).astype(jnp.int32)[:, None]
        outs.append(idx)
        d2 = jnp.where(col == idx, inf, d2)
    out_ref[...] = jnp.concatenate(outs, axis=1)


def _knn(xp, ct, n_valid, rb, k):
    npad, d = xp.shape
    return pl.pallas_call(
        functools.partial(_knn_body, n_valid, k),
        grid=(npad // rb,),
        in_specs=[
            pl.BlockSpec((rb, d), lambda i: (i, 0)),
            pl.BlockSpec((8, npad), lambda i: (0, 0)),
        ],
        out_specs=pl.BlockSpec((rb, k), lambda i: (i, 0)),
        out_shape=jax.ShapeDtypeStruct((npad, k), jnp.int32),
    )(xp, ct)


def _attn_body(scale_den, act, res, k,
               x_ref, xs_ref, wq_ref, bq_ref, wk_ref, bk_ref,
               wv_ref, bv_ref, ws_ref, bs_ref, o_ref):
    nb, d = x_ref.shape
    xb = x_ref[...]
    xs = xs_ref[...]
    q = jnp.dot(xb, wq_ref[...], preferred_element_type=jnp.float32) + bq_ref[...]
    ks = jnp.dot(xs, wk_ref[...], preferred_element_type=jnp.float32) + bk_ref[...]
    vs = jnp.dot(xs, wv_ref[...], preferred_element_type=jnp.float32) + bv_ref[...]
    k3 = ks.reshape(nb, k, d)
    v3 = vs.reshape(nb, k, d)
    logits = jnp.sum(k3 * q[:, None, :], axis=2) / scale_den
    m = jnp.max(logits, axis=1, keepdims=True)
    ex = jnp.exp(logits - m)
    den = jnp.sum(ex, axis=1, keepdims=True)
    alpha = ex / (den + 1e-16)
    agg = jnp.sum(alpha[:, :, None] * v3, axis=1)
    out = agg + jnp.dot(xb, ws_ref[...], preferred_element_type=jnp.float32) + bs_ref[...]
    if res:
        out = out + xb
    if act:
        out = jnp.tanh(out)
    o_ref[...] = out


def _attn(x, xs, wq, bq, wk, bk, wv, bv, ws, bs, ch, act, res, nb, k):
    npad, d = x.shape
    scale_den = float(math.sqrt(float(ch)))
    body = functools.partial(_attn_body, scale_den, act, res, k)
    w_spec = pl.BlockSpec((d, d), lambda i: (0, 0))
    b_spec = pl.BlockSpec((1, d), lambda i: (0, 0))
    return pl.pallas_call(
        body,
        grid=(npad // nb,),
        in_specs=[
            pl.BlockSpec((nb, d), lambda i: (i, 0)),
            pl.BlockSpec((nb * k, d), lambda i: (i, 0)),
            w_spec, b_spec, w_spec, b_spec, w_spec, b_spec, w_spec, b_spec,
        ],
        out_specs=pl.BlockSpec((nb, d), lambda i: (i, 0)),
        out_shape=jax.ShapeDtypeStruct((npad, d), jnp.float32),
    )(x, xs, wq, bq.reshape(1, d), wk, bk.reshape(1, d),
      wv, bv.reshape(1, d), ws, bs.reshape(1, d))


def _sc_gather(table, idx2d):
    npad, d = table.shape
    nrow = idx2d.shape[0]
    ne = nrow * 128
    info = plsc.get_sparse_core_info()
    nw = info.num_cores * info.num_subcores
    rpw = nrow // nw
    mesh = plsc.VectorSubcoreMesh(core_axis_name="c", subcore_axis_name="s")

    @functools.partial(
        pl.kernel,
        mesh=mesh,
        out_type=jax.ShapeDtypeStruct((ne, d), jnp.float32),
        scratch_types=[
            pltpu.VMEM((rpw, 128), jnp.int32),
            pltpu.VMEM((128, d), jnp.float32),
            pltpu.VMEM((128, d), jnp.float32),
            pltpu.SemaphoreType.DMA,
            pltpu.SemaphoreType.DMA,
        ],
    )
    def gk(table_hbm, idx_hbm, out_hbm, idx_v, rows0, rows1, sem0, sem1):
        widx = lax.axis_index("s") * info.num_cores + lax.axis_index("c")
        rbase = widx * rpw
        pltpu.sync_copy(idx_hbm.at[pl.ds(rbase, rpw)], idx_v)
        pltpu.async_copy(table_hbm.at[idx_v.at[0]], rows0, sem0)
        pltpu.async_copy(table_hbm.at[idx_v.at[1]], rows1, sem1)

        def body(s, carry):
            j0 = 2 * s
            pltpu.make_async_copy(table_hbm.at[pl.ds(0, 128)], rows0, sem0).wait()
            pltpu.sync_copy(rows0, out_hbm.at[pl.ds((rbase + j0) * 128, 128)])

            @pl.when(j0 + 2 < rpw)
            def _():
                pltpu.async_copy(table_hbm.at[idx_v.at[j0 + 2]], rows0, sem0)

            pltpu.make_async_copy(table_hbm.at[pl.ds(0, 128)], rows1, sem1).wait()
            pltpu.sync_copy(rows1, out_hbm.at[pl.ds((rbase + j0 + 1) * 128, 128)])

            @pl.when(j0 + 3 < rpw)
            def _():
                pltpu.async_copy(table_hbm.at[idx_v.at[j0 + 3]], rows1, sem1)

            return carry

        lax.fori_loop(0, rpw // 2, body, 0)

    return gk(table, idx2d)


def _pad_w(w, d):
    wp = jnp.zeros((d, d), jnp.float32)
    return wp.at[:, : w.shape[1]].set(w)


def _pad_b(b, d):
    return jnp.zeros((d,), jnp.float32).at[: b.shape[0]].set(b)


def kernel(t, x,
           W1q, b1q, W1k, b1k, W1v, b1v, W1s, b1s,
           W2q, b2q, W2k, b2k, W2v, b2v, W2s, b2s,
           W3q, b3q, W3k, b3k, W3v, b3v, W3s, b3s):
    del t
    d = _D
    xp = jnp.pad(x, ((0, _NPAD - _N), (0, 0)))
    ct = jnp.zeros((8, _NPAD), jnp.float32).at[:3, :_N].set(x[:, :3].T)

    src = _knn(xp, ct, _N, _RB, _K)
    idx2d = src.reshape(_NPAD * _K // 128, 128)

    xs1 = _sc_gather(xp, idx2d)
    h1 = _attn(xp, xs1, W1q, b1q, W1k, b1k, W1v, b1v, W1s, b1s,
               ch=128, act=True, res=False, nb=_AB, k=_K)
    xs2 = _sc_gather(h1, idx2d)
    h2 = _attn(h1, xs2, W2q, b2q, W2k, b2k, W2v, b2v, W2s, b2s,
               ch=128, act=True, res=True, nb=_AB, k=_K)
    xs3 = _sc_gather(h2, idx2d)
    out = _attn(h2, xs3,
                _pad_w(W3q, d), _pad_b(b3q, d), _pad_w(W3k, d), _pad_b(b3k, d),
                _pad_w(W3v, d), _pad_b(b3v, d), _pad_w(W3s, d), _pad_b(b3s, d),
                ch=3, act=False, res=False, nb=_AB, k=_K)
    return out[:_N, :3]
